# baseline (speedup 1.0000x reference)
import jax
import jax.numpy as jnp
from jax import lax
from jax.experimental import pallas as pl
from jax.experimental.pallas import tpu as pltpu
from jax.experimental.pallas import tpu_sc as plsc

B, N, FC = 4, 100000, 200000
BN = B * N
NF = B * FC
NC, NS, L = 2, 16, 16
NW = NC * NS
CH = 1600
CPW = -(-NF // (CH * NW))
NF2 = CH * NW * CPW
INNER = CH // L
ZROWS = 5000
NZ = BN // NS // ZROWS
SPT = BN // NS


def _rsqrt(q):
    yi = jnp.int32(0x5F3759DF) - lax.shift_right_arithmetic(
        lax.bitcast_convert_type(q, jnp.int32), 1)
    y = lax.bitcast_convert_type(yi, jnp.float32)
    h = q * 0.5
    y = y * (1.5 - h * y * y)
    y = y * (1.5 - h * y * y)
    y = y * (1.5 - h * y * y)
    return y


def _sc_body(vx, vy, vz, f0_hbm, f1_hbm, f2_hbm, z_hbm, out_hbm,
             accx, accy, accz,
             idx0, idx1, idx2,
             p00, p01, p02, p10, p11, p12, p20, p21, p22,
             g00, g01, g02, g10, g11, g12, g20, g21, g22,
             tmp, sem_i, sem_g, sem_s):
    c = lax.axis_index("c")
    s = lax.axis_index("s")
    w = c * NS + s
    acc = (accx, accy, accz)
    v_hbm = (vx, vy, vz)
    idx = (idx0, idx1, idx2)
    p = ((p00, p01, p02), (p10, p11, p12), (p20, p21, p22))
    g = ((g00, g01, g02), (g10, g11, g12), (g20, g21, g22))

    pltpu.sync_copy(z_hbm, tmp)
    row0 = s * SPT
    for ax in range(3):
        for j in range(NZ):
            pltpu.sync_copy(tmp, acc[ax].at[pl.ds(row0 + j * ZROWS, ZROWS)])

    plsc.subcore_barrier()

    def _chunk(k, _):
        cid = k * NW + w
        if True:
            base = cid * CH
            for v in range(3):
                pltpu.sync_copy((f0_hbm, f1_hbm, f2_hbm)[v].at[pl.ds(base, CH)],
                                idx[v])
            for v in range(3):
                for ax in range(3):
                    pltpu.sync_copy(v_hbm[ax].at[idx[v]], p[v][ax])

            def _faces(i, _):
                o = i * L
                p0 = [p[0][ax][pl.ds(o, L)] for ax in range(3)]
                p1 = [p[1][ax][pl.ds(o, L)] for ax in range(3)]
                p2 = [p[2][ax][pl.ds(o, L)] for ax in range(3)]
                a = sum((p1[j] - p2[j]) * (p1[j] - p2[j]) for j in range(3))
                b = sum((p2[j] - p0[j]) * (p2[j] - p0[j]) for j in range(3))
                cc = sum((p0[j] - p1[j]) * (p0[j] - p1[j]) for j in range(3))
                t = a + b + cc
                q = t * t - 2.0 * (a * a + b * b + cc * cc)
                r2 = 0.5 * _rsqrt(q)
                c0 = (t - a - a) * r2
                c1 = (t - b - b) * r2
                c2 = (t - cc - cc) * r2
                s0 = c1 + c2
                s1 = c0 + c2
                s2 = c0 + c1
                for j in range(3):
                    g[0][j][pl.ds(o, L)] = c2 * p1[j] + c1 * p2[j] - s0 * p0[j]
                    g[1][j][pl.ds(o, L)] = c0 * p2[j] + c2 * p0[j] - s1 * p1[j]
                    g[2][j][pl.ds(o, L)] = c1 * p0[j] + c0 * p1[j] - s2 * p2[j]
                return 0
            lax.fori_loop(0, INNER, _faces, 0)

            for v in range(3):
                for ax in range(3):
                    pltpu.sync_copy(g[v][ax], acc[ax].at[idx[v]], add=True)
        return 0
    lax.fori_loop(0, CPW, _chunk, 0)

    plsc.subcore_barrier()

    for ax in range(3):
        for j in range(NZ):
            r = row0 + j * ZROWS
            pltpu.sync_copy(acc[ax].at[pl.ds(r, ZROWS)], tmp)
            pltpu.sync_copy(tmp, out_hbm.at[pl.ds((c * 3 + ax) * BN + r, ZROWS)])


_sc_call = pl.kernel(
    _sc_body,
    out_type=jax.ShapeDtypeStruct((NC * 3 * BN,), jnp.float32),
    mesh=plsc.VectorSubcoreMesh(core_axis_name="c", subcore_axis_name="s"),
    scratch_types=(
        [pltpu.VMEM_SHARED((BN,), jnp.float32)] * 3
        + [pltpu.VMEM((CH,), jnp.int32)] * 3
        + [pltpu.VMEM((CH,), jnp.float32)] * 18
        + [pltpu.VMEM((ZROWS,), jnp.float32)]
        + [pltpu.SemaphoreType.DMA] * 3
    ),
)


def _tc_add_body(a_ref, o_ref):
    o_ref[...] = a_ref[0] + a_ref[1]


def kernel(V, F):
    Vf = V.reshape(BN, 3)
    Fi = F.astype(jnp.int32)
    off = (jnp.arange(B, dtype=jnp.int32) * N)[:, None]
    f0 = (Fi[:, :, 0] + off).reshape(-1)
    f1 = (Fi[:, :, 1] + off).reshape(-1)
    f2 = (Fi[:, :, 2] + off).reshape(-1)
    pad = jnp.zeros((NF2 - NF,), jnp.int32)
    f0 = jnp.concatenate([f0, pad])
    f1 = jnp.concatenate([f1, pad])
    f2 = jnp.concatenate([f2, pad])
    z = jnp.zeros((ZROWS,), jnp.float32)
    parts = _sc_call(Vf[:, 0], Vf[:, 1], Vf[:, 2], f0, f1, f2, z)
    summed = pl.pallas_call(
        _tc_add_body,
        out_shape=jax.ShapeDtypeStruct((3 * BN // 128, 128), jnp.float32),
    )(parts.reshape(NC, 3 * BN // 128, 128))
    return summed.reshape(3, BN).T


# speedup vs baseline: 1.8978x; 1.8978x over previous
import jax
import jax.numpy as jnp
from jax import lax
from jax.experimental import pallas as pl
from jax.experimental.pallas import tpu as pltpu
from jax.experimental.pallas import tpu_sc as plsc

B, N, FC = 4, 100000, 200000
BN = B * N
NF = B * FC
NC, NS, L = 2, 16, 16
NW = NC * NS
CH = 3200
NCH = NF // CH
CPW = -(-NCH // NW)
INNER = CH // L
ZROWS = 1000
NZ = BN // NS // ZROWS
SPT = BN // NS


def _rsqrt(q):
    yi = jnp.int32(0x5F3759DF) - lax.shift_right_arithmetic(
        lax.bitcast_convert_type(q, jnp.int32), 1)
    y = lax.bitcast_convert_type(yi, jnp.float32)
    h = q * 0.5
    y = y * (1.5 - h * y * y)
    y = y * (1.5 - h * y * y)
    y = y * (1.5 - h * y * y)
    return y


def _sc_body(vx, vy, vz, f0_hbm, f1_hbm, f2_hbm, z_hbm, out_hbm,
             accx, accy, accz, idx, px, py, pz, tmp):
    c = lax.axis_index("c")
    s = lax.axis_index("s")
    w = c * NS + s
    acc = (accx, accy, accz)
    pxyz = (px, py, pz)

    pltpu.sync_copy(z_hbm, tmp)
    row0 = s * SPT
    for ax in range(3):
        for j in range(NZ):
            pltpu.sync_copy(tmp, acc[ax].at[pl.ds(row0 + j * ZROWS, ZROWS)])

    plsc.subcore_barrier()

    def _chunk(k, _):
        cid = k * NW + w

        @pl.when(cid < NCH)
        def _():
            base = cid * CH
            for v in range(3):
                pltpu.sync_copy((f0_hbm, f1_hbm, f2_hbm)[v].at[pl.ds(base, CH)],
                                idx.at[pl.ds(v * CH, CH)])
            for ax in range(3):
                pltpu.sync_copy((vx, vy, vz)[ax].at[idx], pxyz[ax])

            def _faces(i, _):
                o = i * L
                p0 = [pxyz[ax][pl.ds(o, L)] for ax in range(3)]
                p1 = [pxyz[ax][pl.ds(CH + o, L)] for ax in range(3)]
                p2 = [pxyz[ax][pl.ds(2 * CH + o, L)] for ax in range(3)]
                a = sum((p1[j] - p2[j]) * (p1[j] - p2[j]) for j in range(3))
                b = sum((p2[j] - p0[j]) * (p2[j] - p0[j]) for j in range(3))
                cc = sum((p0[j] - p1[j]) * (p0[j] - p1[j]) for j in range(3))
                t = a + b + cc
                q = t * t - 2.0 * (a * a + b * b + cc * cc)
                r2 = 0.5 * _rsqrt(q)
                c0 = (t - a - a) * r2
                c1 = (t - b - b) * r2
                c2 = (t - cc - cc) * r2
                s0 = c1 + c2
                s1 = c0 + c2
                s2 = c0 + c1
                for j in range(3):
                    pxyz[j][pl.ds(o, L)] = (c2 * p1[j] + c1 * p2[j]
                                            - s0 * p0[j])
                    pxyz[j][pl.ds(CH + o, L)] = (c0 * p2[j] + c2 * p0[j]
                                                 - s1 * p1[j])
                    pxyz[j][pl.ds(2 * CH + o, L)] = (c1 * p0[j] + c0 * p1[j]
                                                     - s2 * p2[j])
                return 0
            lax.fori_loop(0, INNER, _faces, 0)

            for ax in range(3):
                pltpu.sync_copy(pxyz[ax], acc[ax].at[idx], add=True)
        return 0
    lax.fori_loop(0, CPW, _chunk, 0)

    plsc.subcore_barrier()

    for ax in range(3):
        for j in range(NZ):
            r = row0 + j * ZROWS
            pltpu.sync_copy(acc[ax].at[pl.ds(r, ZROWS)], tmp)
            pltpu.sync_copy(tmp, out_hbm.at[pl.ds((c * 3 + ax) * BN + r, ZROWS)])


_sc_call = pl.kernel(
    _sc_body,
    out_type=jax.ShapeDtypeStruct((NC * 3 * BN,), jnp.float32),
    mesh=plsc.VectorSubcoreMesh(core_axis_name="c", subcore_axis_name="s"),
    scratch_types=(
        [pltpu.VMEM_SHARED((BN,), jnp.float32)] * 3
        + [pltpu.VMEM((3 * CH,), jnp.int32)]
        + [pltpu.VMEM((3 * CH,), jnp.float32)] * 3
        + [pltpu.VMEM((ZROWS,), jnp.float32)]
    ),
)


def _tc_add_body(a_ref, o_ref):
    o_ref[...] = a_ref[0] + a_ref[1]


def kernel(V, F):
    Vf = V.reshape(BN, 3)
    Fi = F.astype(jnp.int32)
    off = (jnp.arange(B, dtype=jnp.int32) * N)[:, None]
    f0 = (Fi[:, :, 0] + off).reshape(-1)
    f1 = (Fi[:, :, 1] + off).reshape(-1)
    f2 = (Fi[:, :, 2] + off).reshape(-1)
    z = jnp.zeros((ZROWS,), jnp.float32)
    parts = _sc_call(Vf[:, 0], Vf[:, 1], Vf[:, 2], f0, f1, f2, z)
    summed = pl.pallas_call(
        _tc_add_body,
        out_shape=jax.ShapeDtypeStruct((3 * BN // 128, 128), jnp.float32),
    )(parts.reshape(NC, 3 * BN // 128, 128))
    return summed.reshape(3, BN).T


# speedup vs baseline: 2.0275x; 1.0684x over previous
import jax
import jax.numpy as jnp
from jax import lax
from jax.experimental import pallas as pl
from jax.experimental.pallas import tpu as pltpu
from jax.experimental.pallas import tpu_sc as plsc

B, N, FC = 4, 100000, 200000
BN = B * N
NF = B * FC
NC, NS, L = 2, 16, 16
NW = NC * NS
CH = 3200
NCH = NF // CH
CPW = -(-NCH // NW)
INNER = CH // L
ZROWS = 1000
NZ = BN // NS // ZROWS
SPT = BN // NS


def _rsqrt(q):
    yi = jnp.int32(0x5F3759DF) - lax.shift_right_arithmetic(
        lax.bitcast_convert_type(q, jnp.int32), 1)
    y = lax.bitcast_convert_type(yi, jnp.float32)
    h = q * 0.5
    y = y * (1.5 - h * y * y)
    y = y * (1.5 - h * y * y)
    y = y * (1.5 - h * y * y)
    return y


def _sc_body(vx, vy, vz, f0_hbm, f1_hbm, f2_hbm, z_hbm, out_hbm,
             accx, accy, accz, idx, px, py, pz, tmp, sem_i, sem_g, sem_s):
    c = lax.axis_index("c")
    s = lax.axis_index("s")
    w = c * NS + s
    acc = (accx, accy, accz)
    pxyz = (px, py, pz)

    pltpu.sync_copy(z_hbm, tmp)
    row0 = s * SPT
    for ax in range(3):
        for j in range(NZ):
            pltpu.sync_copy(tmp, acc[ax].at[pl.ds(row0 + j * ZROWS, ZROWS)])

    plsc.subcore_barrier()

    def _chunk(k, _):
        cid = k * NW + w

        @pl.when(cid < NCH)
        def _():
            base = cid * CH
            d_i = [pltpu.async_copy(
                (f0_hbm, f1_hbm, f2_hbm)[v].at[pl.ds(base, CH)],
                idx.at[pl.ds(v * CH, CH)], sem_i) for v in range(3)]
            for d in d_i:
                d.wait()
            d_g = [pltpu.async_copy((vx, vy, vz)[ax].at[idx], pxyz[ax], sem_g)
                   for ax in range(3)]
            for d in d_g:
                d.wait()

            def _faces(i, _):
                o = i * L
                p0 = [pxyz[ax][pl.ds(o, L)] for ax in range(3)]
                p1 = [pxyz[ax][pl.ds(CH + o, L)] for ax in range(3)]
                p2 = [pxyz[ax][pl.ds(2 * CH + o, L)] for ax in range(3)]
                e01 = [p0[j] - p1[j] for j in range(3)]
                e12 = [p1[j] - p2[j] for j in range(3)]
                e20 = [p2[j] - p0[j] for j in range(3)]
                a = sum(e12[j] * e12[j] for j in range(3))
                b = sum(e20[j] * e20[j] for j in range(3))
                cc = sum(e01[j] * e01[j] for j in range(3))
                t = a + b + cc
                q = t * t - 2.0 * (a * a + b * b + cc * cc)
                r2 = 0.5 * _rsqrt(q)
                c0 = (t - a - a) * r2
                c1 = (t - b - b) * r2
                c2 = (t - cc - cc) * r2
                for j in range(3):
                    t0 = c0 * e12[j]
                    t1 = c1 * e20[j]
                    t2 = c2 * e01[j]
                    pxyz[j][pl.ds(o, L)] = t1 - t2
                    pxyz[j][pl.ds(CH + o, L)] = t2 - t0
                    pxyz[j][pl.ds(2 * CH + o, L)] = t0 - t1
                return 0
            lax.fori_loop(0, INNER, _faces, 0)

            d_s = [pltpu.async_copy(pxyz[ax], acc[ax].at[idx], sem_s,
                                    add=True) for ax in range(3)]
            for d in d_s:
                d.wait()
        return 0
    lax.fori_loop(0, CPW, _chunk, 0)

    plsc.subcore_barrier()

    for ax in range(3):
        for j in range(NZ):
            r = row0 + j * ZROWS
            pltpu.sync_copy(acc[ax].at[pl.ds(r, ZROWS)], tmp)
            pltpu.sync_copy(tmp, out_hbm.at[pl.ds((c * 3 + ax) * BN + r, ZROWS)])


_sc_call = pl.kernel(
    _sc_body,
    out_type=jax.ShapeDtypeStruct((NC * 3 * BN,), jnp.float32),
    mesh=plsc.VectorSubcoreMesh(core_axis_name="c", subcore_axis_name="s"),
    scratch_types=(
        [pltpu.VMEM_SHARED((BN,), jnp.float32)] * 3
        + [pltpu.VMEM((3 * CH,), jnp.int32)]
        + [pltpu.VMEM((3 * CH,), jnp.float32)] * 3
        + [pltpu.VMEM((ZROWS,), jnp.float32)]
        + [pltpu.SemaphoreType.DMA] * 3
    ),
)


def _tc_add_body(a_ref, o_ref):
    o_ref[...] = a_ref[0] + a_ref[1]


def kernel(V, F):
    Vf = V.reshape(BN, 3)
    Fi = F.astype(jnp.int32)
    off = (jnp.arange(B, dtype=jnp.int32) * N)[:, None]
    f0 = (Fi[:, :, 0] + off).reshape(-1)
    f1 = (Fi[:, :, 1] + off).reshape(-1)
    f2 = (Fi[:, :, 2] + off).reshape(-1)
    z = jnp.zeros((ZROWS,), jnp.float32)
    parts = _sc_call(Vf[:, 0], Vf[:, 1], Vf[:, 2], f0, f1, f2, z)
    summed = pl.pallas_call(
        _tc_add_body,
        out_shape=jax.ShapeDtypeStruct((3 * BN // 128, 128), jnp.float32),
    )(parts.reshape(NC, 3 * BN // 128, 128))
    return summed.reshape(3, BN).T


# speedup vs baseline: 2.2584x; 1.1139x over previous
import jax
import jax.numpy as jnp
from jax import lax
from jax.experimental import pallas as pl
from jax.experimental.pallas import tpu as pltpu
from jax.experimental.pallas import tpu_sc as plsc

B, N, FC = 4, 100000, 200000
BN = B * N
NF = B * FC
NC, NS, L = 2, 16, 16
NW = NC * NS
CH = 1600
NCH = NF // (2 * CH)
CPW = -(-NCH // NW)
INNER = CH // L
ZROWS = 1000
NZ = BN // NS // ZROWS
SPT = BN // NS


def _rsqrt(q):
    yi = jnp.int32(0x5F3759DF) - lax.shift_right_arithmetic(
        lax.bitcast_convert_type(q, jnp.int32), 1)
    y = lax.bitcast_convert_type(yi, jnp.float32)
    h = q * 0.5
    y = y * (1.5 - h * y * y)
    y = y * (1.5 - h * y * y)
    y = y * (1.5 - h * y * y)
    return y


def _sc_body(vx, vy, vz, f0_hbm, f1_hbm, f2_hbm, z_hbm, out_hbm,
             accx, accy, accz, idxA, pxA, pyA, pzA, idxB, pxB, pyB, pzB, tmp,
             sem_i, sem_ga, sem_gb, sem_sa, sem_sb):
    c = lax.axis_index("c")
    s = lax.axis_index("s")
    w = c * NS + s
    acc = (accx, accy, accz)

    pltpu.sync_copy(z_hbm, tmp)
    row0 = s * SPT
    for ax in range(3):
        for j in range(NZ):
            pltpu.sync_copy(tmp, acc[ax].at[pl.ds(row0 + j * ZROWS, ZROWS)])

    plsc.subcore_barrier()

    def _compute(pxyz):
        def _faces(i, _):
            o = i * L
            p0 = [pxyz[ax][pl.ds(o, L)] for ax in range(3)]
            p1 = [pxyz[ax][pl.ds(CH + o, L)] for ax in range(3)]
            p2 = [pxyz[ax][pl.ds(2 * CH + o, L)] for ax in range(3)]
            e01 = [p0[j] - p1[j] for j in range(3)]
            e12 = [p1[j] - p2[j] for j in range(3)]
            e20 = [p2[j] - p0[j] for j in range(3)]
            a = sum(e12[j] * e12[j] for j in range(3))
            b = sum(e20[j] * e20[j] for j in range(3))
            cc = sum(e01[j] * e01[j] for j in range(3))
            t = a + b + cc
            q = t * t - 2.0 * (a * a + b * b + cc * cc)
            r2 = 0.5 * _rsqrt(q)
            c0 = (t - a - a) * r2
            c1 = (t - b - b) * r2
            c2 = (t - cc - cc) * r2
            for j in range(3):
                t0 = c0 * e12[j]
                t1 = c1 * e20[j]
                t2 = c2 * e01[j]
                pxyz[j][pl.ds(o, L)] = t1 - t2
                pxyz[j][pl.ds(CH + o, L)] = t2 - t0
                pxyz[j][pl.ds(2 * CH + o, L)] = t0 - t1
            return 0
        lax.fori_loop(0, INNER, _faces, 0)

    def _chunk(k, _):
        cid = k * NW + w

        @pl.when(cid < NCH)
        def _():
            baseA = cid * (2 * CH)
            baseB = baseA + CH
            pA = (pxA, pyA, pzA)
            pB = (pxB, pyB, pzB)
            d_ia = [pltpu.async_copy(
                (f0_hbm, f1_hbm, f2_hbm)[v].at[pl.ds(baseA, CH)],
                idxA.at[pl.ds(v * CH, CH)], sem_i) for v in range(3)]
            d_ib = [pltpu.async_copy(
                (f0_hbm, f1_hbm, f2_hbm)[v].at[pl.ds(baseB, CH)],
                idxB.at[pl.ds(v * CH, CH)], sem_i) for v in range(3)]
            for d in d_ia:
                d.wait()
            d_ga = [pltpu.async_copy((vx, vy, vz)[ax].at[idxA], pA[ax],
                                     sem_ga) for ax in range(3)]
            for d in d_ib:
                d.wait()
            d_gb = [pltpu.async_copy((vx, vy, vz)[ax].at[idxB], pB[ax],
                                     sem_gb) for ax in range(3)]
            for d in d_ga:
                d.wait()
            _compute(pA)
            d_sa = [pltpu.async_copy(pA[ax], acc[ax].at[idxA], sem_sa,
                                     add=True) for ax in range(3)]
            for d in d_gb:
                d.wait()
            _compute(pB)
            d_sb = [pltpu.async_copy(pB[ax], acc[ax].at[idxB], sem_sb,
                                     add=True) for ax in range(3)]
            for d in d_sa:
                d.wait()
            for d in d_sb:
                d.wait()
        return 0
    lax.fori_loop(0, CPW, _chunk, 0)

    plsc.subcore_barrier()

    for ax in range(3):
        for j in range(NZ):
            r = row0 + j * ZROWS
            pltpu.sync_copy(acc[ax].at[pl.ds(r, ZROWS)], tmp)
            pltpu.sync_copy(tmp, out_hbm.at[pl.ds((c * 3 + ax) * BN + r, ZROWS)])


_sc_call = pl.kernel(
    _sc_body,
    out_type=jax.ShapeDtypeStruct((NC * 3 * BN,), jnp.float32),
    mesh=plsc.VectorSubcoreMesh(core_axis_name="c", subcore_axis_name="s"),
    scratch_types=(
        [pltpu.VMEM_SHARED((BN,), jnp.float32)] * 3
        + [pltpu.VMEM((3 * CH,), jnp.int32)]
        + [pltpu.VMEM((3 * CH,), jnp.float32)] * 3
        + [pltpu.VMEM((3 * CH,), jnp.int32)]
        + [pltpu.VMEM((3 * CH,), jnp.float32)] * 3
        + [pltpu.VMEM((ZROWS,), jnp.float32)]
        + [pltpu.SemaphoreType.DMA] * 5
    ),
)


def _tc_add_body(a_ref, o_ref):
    o_ref[...] = a_ref[0] + a_ref[1]


def kernel(V, F):
    Vf = V.reshape(BN, 3)
    Fi = F.astype(jnp.int32)
    off = (jnp.arange(B, dtype=jnp.int32) * N)[:, None]
    f0 = (Fi[:, :, 0] + off).reshape(-1)
    f1 = (Fi[:, :, 1] + off).reshape(-1)
    f2 = (Fi[:, :, 2] + off).reshape(-1)
    z = jnp.zeros((ZROWS,), jnp.float32)
    parts = _sc_call(Vf[:, 0], Vf[:, 1], Vf[:, 2], f0, f1, f2, z)
    summed = pl.pallas_call(
        _tc_add_body,
        out_shape=jax.ShapeDtypeStruct((3 * BN // 128, 128), jnp.float32),
    )(parts.reshape(NC, 3 * BN // 128, 128))
    return summed.reshape(3, BN).T


# speedup vs baseline: 2.3940x; 1.0601x over previous
import jax
import jax.numpy as jnp
from jax import lax
from jax.experimental import pallas as pl
from jax.experimental.pallas import tpu as pltpu
from jax.experimental.pallas import tpu_sc as plsc

B, N, FC = 4, 100000, 200000
BN = B * N
NF = B * FC
NC, NS, L = 2, 16, 16
NW = NC * NS
CH = 800
NSUB = 4
NCH = NF // (NSUB * CH)
CPW = -(-NCH // NW)
INNER = CH // L
ZROWS = 1000
NZ = BN // NS // ZROWS
SPT = BN // NS


def _rsqrt(q):
    yi = jnp.int32(0x5F3759DF) - lax.shift_right_arithmetic(
        lax.bitcast_convert_type(q, jnp.int32), 1)
    y = lax.bitcast_convert_type(yi, jnp.float32)
    h = q * 0.5
    y = y * (1.5 - h * y * y)
    y = y * (1.5 - h * y * y)
    y = y * (1.5 - h * y * y)
    return y


def _sc_body(vx, vy, vz, f0_hbm, f1_hbm, f2_hbm, z_hbm, out_hbm,
             accx, accy, accz,
             idx0, px0, py0, pz0, idx1, px1, py1, pz1,
             idx2, px2, py2, pz2, idx3, px3, py3, pz3, tmp,
             sem_i, sem_g0, sem_g1, sem_g2, sem_g3,
             sem_s0, sem_s1, sem_s2, sem_s3):
    c = lax.axis_index("c")
    s = lax.axis_index("s")
    w = c * NS + s
    acc = (accx, accy, accz)

    pltpu.sync_copy(z_hbm, tmp)
    row0 = s * SPT
    for ax in range(3):
        for j in range(NZ):
            pltpu.sync_copy(tmp, acc[ax].at[pl.ds(row0 + j * ZROWS, ZROWS)])

    plsc.subcore_barrier()

    def _compute(pxyz):
        def _faces(i, _):
            o = i * L
            p0 = [pxyz[ax][pl.ds(o, L)] for ax in range(3)]
            p1 = [pxyz[ax][pl.ds(CH + o, L)] for ax in range(3)]
            p2 = [pxyz[ax][pl.ds(2 * CH + o, L)] for ax in range(3)]
            e01 = [p0[j] - p1[j] for j in range(3)]
            e12 = [p1[j] - p2[j] for j in range(3)]
            e20 = [p2[j] - p0[j] for j in range(3)]
            a = sum(e12[j] * e12[j] for j in range(3))
            b = sum(e20[j] * e20[j] for j in range(3))
            cc = sum(e01[j] * e01[j] for j in range(3))
            t = a + b + cc
            q = t * t - 2.0 * (a * a + b * b + cc * cc)
            r2 = 0.5 * _rsqrt(q)
            c0 = (t - a - a) * r2
            c1 = (t - b - b) * r2
            c2 = (t - cc - cc) * r2
            for j in range(3):
                t0 = c0 * e12[j]
                t1 = c1 * e20[j]
                t2 = c2 * e01[j]
                pxyz[j][pl.ds(o, L)] = t1 - t2
                pxyz[j][pl.ds(CH + o, L)] = t2 - t0
                pxyz[j][pl.ds(2 * CH + o, L)] = t0 - t1
            return 0
        lax.fori_loop(0, INNER, _faces, 0)

    idxq = (idx0, idx1, idx2, idx3)
    pq = ((px0, py0, pz0), (px1, py1, pz1), (px2, py2, pz2), (px3, py3, pz3))
    sgq = (sem_g0, sem_g1, sem_g2, sem_g3)
    ssq = (sem_s0, sem_s1, sem_s2, sem_s3)

    def _chunk(k, _):
        cid = k * NW + w

        @pl.when(cid < NCH)
        def _():
            base = cid * (NSUB * CH)
            d_i = [[pltpu.async_copy(
                (f0_hbm, f1_hbm, f2_hbm)[v].at[pl.ds(base + qq * CH, CH)],
                idxq[qq].at[pl.ds(v * CH, CH)], sem_i) for v in range(3)]
                for qq in range(NSUB)]
            d_g = []
            for qq in range(NSUB):
                for d in d_i[qq]:
                    d.wait()
                d_g.append([pltpu.async_copy(
                    (vx, vy, vz)[ax].at[idxq[qq]], pq[qq][ax], sgq[qq])
                    for ax in range(3)])
            d_s = []
            for qq in range(NSUB):
                for d in d_g[qq]:
                    d.wait()
                _compute(pq[qq])
                d_s.append([pltpu.async_copy(
                    pq[qq][ax], acc[ax].at[idxq[qq]], ssq[qq], add=True)
                    for ax in range(3)])
            for qq in range(NSUB):
                for d in d_s[qq]:
                    d.wait()
        return 0
    lax.fori_loop(0, CPW, _chunk, 0)

    plsc.subcore_barrier()

    for ax in range(3):
        for j in range(NZ):
            r = row0 + j * ZROWS
            pltpu.sync_copy(acc[ax].at[pl.ds(r, ZROWS)], tmp)
            pltpu.sync_copy(tmp, out_hbm.at[pl.ds((c * 3 + ax) * BN + r, ZROWS)])


_sc_call = pl.kernel(
    _sc_body,
    out_type=jax.ShapeDtypeStruct((NC * 3 * BN,), jnp.float32),
    mesh=plsc.VectorSubcoreMesh(core_axis_name="c", subcore_axis_name="s"),
    scratch_types=(
        [pltpu.VMEM_SHARED((BN,), jnp.float32)] * 3
        + ([pltpu.VMEM((3 * CH,), jnp.int32)]
           + [pltpu.VMEM((3 * CH,), jnp.float32)] * 3) * 4
        + [pltpu.VMEM((ZROWS,), jnp.float32)]
        + [pltpu.SemaphoreType.DMA] * 9
    ),
)


def _tc_add_body(a_ref, o_ref):
    o_ref[...] = a_ref[0] + a_ref[1]


def kernel(V, F):
    Vf = V.reshape(BN, 3)
    Fi = F.astype(jnp.int32)
    off = (jnp.arange(B, dtype=jnp.int32) * N)[:, None]
    f0 = (Fi[:, :, 0] + off).reshape(-1)
    f1 = (Fi[:, :, 1] + off).reshape(-1)
    f2 = (Fi[:, :, 2] + off).reshape(-1)
    z = jnp.zeros((ZROWS,), jnp.float32)
    parts = _sc_call(Vf[:, 0], Vf[:, 1], Vf[:, 2], f0, f1, f2, z)
    summed = pl.pallas_call(
        _tc_add_body,
        out_shape=jax.ShapeDtypeStruct((3 * BN // 128, 128), jnp.float32),
    )(parts.reshape(NC, 3 * BN // 128, 128))
    return summed.reshape(3, BN).T


# speedup vs baseline: 2.5367x; 1.0596x over previous
import jax
import jax.numpy as jnp
from jax import lax
from jax.experimental import pallas as pl
from jax.experimental.pallas import tpu as pltpu
from jax.experimental.pallas import tpu_sc as plsc

B, N, FC = 4, 100000, 200000
BN = B * N
NF = B * FC
NC, NS, L = 2, 16, 16
NW = NC * NS
CH = 800
NSUB = 4
NCH = NF // (NSUB * CH)
CPW = -(-NCH // NW)
INNER = CH // L
ZROWS = 1000
NZ = BN // NS // ZROWS
SPT = BN // NS


def _rsqrt(q):
    yi = jnp.int32(0x5F3759DF) - lax.shift_right_arithmetic(
        lax.bitcast_convert_type(q, jnp.int32), 1)
    y = lax.bitcast_convert_type(yi, jnp.float32)
    h = q * 0.5
    y = y * (1.5 - h * y * y)
    y = y * (1.5 - h * y * y)
    y = y * (1.5 - h * y * y)
    return y


def _sc_body(vt_hbm, fall_hbm, z_hbm, out_hbm,
             accx, accy, accz,
             idx0, px0, py0, pz0, idx1, px1, py1, pz1,
             idx2, px2, py2, pz2, idx3, px3, py3, pz3, tmp,
             sem_i, sem_g0, sem_g1, sem_g2, sem_g3,
             sem_s0, sem_s1, sem_s2, sem_s3):
    c = lax.axis_index("c")
    s = lax.axis_index("s")
    w = c * NS + s
    acc = (accx, accy, accz)

    pltpu.sync_copy(z_hbm, tmp)
    row0 = s * SPT
    for ax in range(3):
        for j in range(NZ):
            pltpu.sync_copy(tmp, acc[ax].at[pl.ds(row0 + j * ZROWS, ZROWS)])

    plsc.subcore_barrier()

    def _compute(pxyz):
        def _faces(i, _):
            o = i * L
            p0 = [pxyz[ax][pl.ds(o, L)] for ax in range(3)]
            p1 = [pxyz[ax][pl.ds(CH + o, L)] for ax in range(3)]
            p2 = [pxyz[ax][pl.ds(2 * CH + o, L)] for ax in range(3)]
            e01 = [p0[j] - p1[j] for j in range(3)]
            e12 = [p1[j] - p2[j] for j in range(3)]
            e20 = [p2[j] - p0[j] for j in range(3)]
            a = sum(e12[j] * e12[j] for j in range(3))
            b = sum(e20[j] * e20[j] for j in range(3))
            cc = sum(e01[j] * e01[j] for j in range(3))
            t = a + b + cc
            q = t * t - 2.0 * (a * a + b * b + cc * cc)
            r2 = 0.5 * _rsqrt(q)
            c0 = (t - a - a) * r2
            c1 = (t - b - b) * r2
            c2 = (t - cc - cc) * r2
            for j in range(3):
                t0 = c0 * e12[j]
                t1 = c1 * e20[j]
                t2 = c2 * e01[j]
                pxyz[j][pl.ds(o, L)] = t1 - t2
                pxyz[j][pl.ds(CH + o, L)] = t2 - t0
                pxyz[j][pl.ds(2 * CH + o, L)] = t0 - t1
            return 0
        lax.fori_loop(0, INNER, _faces, 0)

    idxq = (idx0, idx1, idx2, idx3)
    pq = ((px0, py0, pz0), (px1, py1, pz1), (px2, py2, pz2), (px3, py3, pz3))
    sgq = (sem_g0, sem_g1, sem_g2, sem_g3)
    ssq = (sem_s0, sem_s1, sem_s2, sem_s3)

    def _chunk(k, _):
        cid = k * NW + w

        @pl.when(cid < NCH)
        def _():
            base = cid * (NSUB * CH)
            d_i = [[pltpu.async_copy(
                fall_hbm.at[pl.ds(v * NF + base + qq * CH, CH)],
                idxq[qq].at[pl.ds(v * CH, CH)], sem_i) for v in range(3)]
                for qq in range(NSUB)]
            d_g = []
            for qq in range(NSUB):
                for d in d_i[qq]:
                    d.wait()
                d_g.append([pltpu.async_copy(
                    vt_hbm.at[pl.ds(ax * BN, BN)].at[idxq[qq]],
                    pq[qq][ax], sgq[qq])
                    for ax in range(3)])
            d_s = []
            for qq in range(NSUB):
                for d in d_g[qq]:
                    d.wait()
                _compute(pq[qq])
                d_s.append([pltpu.async_copy(
                    pq[qq][ax], acc[ax].at[idxq[qq]], ssq[qq], add=True)
                    for ax in range(3)])
            for qq in range(NSUB):
                for d in d_s[qq]:
                    d.wait()
        return 0
    lax.fori_loop(0, CPW, _chunk, 0)

    plsc.subcore_barrier()

    for ax in range(3):
        for j in range(NZ):
            r = row0 + j * ZROWS
            pltpu.sync_copy(acc[ax].at[pl.ds(r, ZROWS)], tmp)
            pltpu.sync_copy(tmp, out_hbm.at[pl.ds((c * 3 + ax) * BN + r, ZROWS)])


_sc_call = pl.kernel(
    _sc_body,
    out_type=jax.ShapeDtypeStruct((NC * 3 * BN,), jnp.float32),
    mesh=plsc.VectorSubcoreMesh(core_axis_name="c", subcore_axis_name="s"),
    scratch_types=(
        [pltpu.VMEM_SHARED((BN,), jnp.float32)] * 3
        + ([pltpu.VMEM((3 * CH,), jnp.int32)]
           + [pltpu.VMEM((3 * CH,), jnp.float32)] * 3) * 4
        + [pltpu.VMEM((ZROWS,), jnp.float32)]
        + [pltpu.SemaphoreType.DMA] * 9
    ),
)


def _tc_add_body(a_ref, o_ref):
    o_ref[...] = a_ref[0] + a_ref[1]


def kernel(V, F):
    vt = V.reshape(BN, 3).T.reshape(3 * BN)
    off = (jnp.arange(B, dtype=jnp.int32) * N)[:, None, None]
    fall = (F.astype(jnp.int32) + off).transpose(2, 0, 1).reshape(3 * NF)
    z = jnp.zeros((ZROWS,), jnp.float32)
    parts = _sc_call(vt, fall, z)
    summed = pl.pallas_call(
        _tc_add_body,
        out_shape=jax.ShapeDtypeStruct((3 * BN // 128, 128), jnp.float32),
    )(parts.reshape(NC, 3 * BN // 128, 128))
    return summed.reshape(3, BN).T


# speedup vs baseline: 2.8203x; 1.1118x over previous
import jax
import jax.numpy as jnp
from jax import lax
from jax.experimental import pallas as pl
from jax.experimental.pallas import tpu as pltpu
from jax.experimental.pallas import tpu_sc as plsc

B, N, FC = 4, 100000, 200000
BN = B * N
NF = B * FC
NC, NS, L = 2, 16, 16
NW = NC * NS
CH = 400
NSUB = 8
NCH = NF // (NSUB * CH)
CPW = -(-NCH // NW)
INNER = CH // L
ZROWS = 5000
NZ = BN // NS // ZROWS
SPT = BN // NS


def _rsqrt(q):
    yi = jnp.int32(0x5F3759DF) - lax.shift_right_arithmetic(
        lax.bitcast_convert_type(q, jnp.int32), 1)
    y = lax.bitcast_convert_type(yi, jnp.float32)
    h = q * 0.5
    y = y * (1.5 - h * y * y)
    y = y * (1.5 - h * y * y)
    y = y * (1.5 - h * y * y)
    return y


def _sc_body(vt_hbm, fall_hbm, z_hbm, out_hbm,
             accx, accy, accz,
             idx0, px0, py0, pz0, idx1, px1, py1, pz1,
             idx2, px2, py2, pz2, idx3, px3, py3, pz3,
             idx4, px4, py4, pz4, idx5, px5, py5, pz5,
             idx6, px6, py6, pz6, idx7, px7, py7, pz7, tmp,
             sem_i, sem_g0, sem_g1, sem_g2, sem_g3,
             sem_g4, sem_g5, sem_g6, sem_g7,
             sem_s0, sem_s1, sem_s2, sem_s3,
             sem_s4, sem_s5, sem_s6, sem_s7):
    c = lax.axis_index("c")
    s = lax.axis_index("s")
    w = c * NS + s
    acc = (accx, accy, accz)

    pltpu.sync_copy(z_hbm, tmp)
    row0 = s * SPT
    for ax in range(3):
        for j in range(NZ):
            pltpu.sync_copy(tmp, acc[ax].at[pl.ds(row0 + j * ZROWS, ZROWS)])

    plsc.subcore_barrier()

    def _compute(pxyz):
        def _faces(i, _):
            o = i * L
            p0 = [pxyz[ax][pl.ds(o, L)] for ax in range(3)]
            p1 = [pxyz[ax][pl.ds(CH + o, L)] for ax in range(3)]
            p2 = [pxyz[ax][pl.ds(2 * CH + o, L)] for ax in range(3)]
            e01 = [p0[j] - p1[j] for j in range(3)]
            e12 = [p1[j] - p2[j] for j in range(3)]
            e20 = [p2[j] - p0[j] for j in range(3)]
            a = sum(e12[j] * e12[j] for j in range(3))
            b = sum(e20[j] * e20[j] for j in range(3))
            cc = sum(e01[j] * e01[j] for j in range(3))
            t = a + b + cc
            q = t * t - 2.0 * (a * a + b * b + cc * cc)
            r2 = 0.5 * _rsqrt(q)
            c0 = (t - a - a) * r2
            c1 = (t - b - b) * r2
            c2 = (t - cc - cc) * r2
            for j in range(3):
                t0 = c0 * e12[j]
                t1 = c1 * e20[j]
                t2 = c2 * e01[j]
                pxyz[j][pl.ds(o, L)] = t1 - t2
                pxyz[j][pl.ds(CH + o, L)] = t2 - t0
                pxyz[j][pl.ds(2 * CH + o, L)] = t0 - t1
            return 0
        lax.fori_loop(0, INNER, _faces, 0)

    idxq = (idx0, idx1, idx2, idx3, idx4, idx5, idx6, idx7)
    pq = ((px0, py0, pz0), (px1, py1, pz1), (px2, py2, pz2), (px3, py3, pz3),
          (px4, py4, pz4), (px5, py5, pz5), (px6, py6, pz6), (px7, py7, pz7))
    sgq = (sem_g0, sem_g1, sem_g2, sem_g3, sem_g4, sem_g5, sem_g6, sem_g7)
    ssq = (sem_s0, sem_s1, sem_s2, sem_s3, sem_s4, sem_s5, sem_s6, sem_s7)

    def _chunk(k, _):
        cid = k * NW + w

        @pl.when(cid < NCH)
        def _():
            base = cid * (NSUB * CH)
            d_i = [[pltpu.async_copy(
                fall_hbm.at[pl.ds(v * NF + base + qq * CH, CH)],
                idxq[qq].at[pl.ds(v * CH, CH)], sem_i) for v in range(3)]
                for qq in range(NSUB)]
            d_g = []
            for qq in range(NSUB):
                for d in d_i[qq]:
                    d.wait()
                d_g.append([pltpu.async_copy(
                    vt_hbm.at[pl.ds(ax * BN, BN)].at[idxq[qq]],
                    pq[qq][ax], sgq[qq])
                    for ax in range(3)])
            d_s = []
            for qq in range(NSUB):
                for d in d_g[qq]:
                    d.wait()
                _compute(pq[qq])
                d_s.append([pltpu.async_copy(
                    pq[qq][ax], acc[ax].at[idxq[qq]], ssq[qq], add=True)
                    for ax in range(3)])
            for qq in range(NSUB):
                for d in d_s[qq]:
                    d.wait()
        return 0
    lax.fori_loop(0, CPW, _chunk, 0)

    plsc.subcore_barrier()

    for ax in range(3):
        for j in range(NZ):
            r = row0 + j * ZROWS
            pltpu.sync_copy(acc[ax].at[pl.ds(r, ZROWS)], tmp)
            pltpu.sync_copy(tmp, out_hbm.at[pl.ds((c * 3 + ax) * BN + r, ZROWS)])


_sc_call = pl.kernel(
    _sc_body,
    out_type=jax.ShapeDtypeStruct((NC * 3 * BN,), jnp.float32),
    mesh=plsc.VectorSubcoreMesh(core_axis_name="c", subcore_axis_name="s"),
    scratch_types=(
        [pltpu.VMEM_SHARED((BN,), jnp.float32)] * 3
        + ([pltpu.VMEM((3 * CH,), jnp.int32)]
           + [pltpu.VMEM((3 * CH,), jnp.float32)] * 3) * 8
        + [pltpu.VMEM((ZROWS,), jnp.float32)]
        + [pltpu.SemaphoreType.DMA] * 17
    ),
)


def _tc_add_body(a_ref, o_ref):
    o_ref[...] = a_ref[0] + a_ref[1]


def kernel(V, F):
    vt = V.reshape(BN, 3).T.reshape(3 * BN)
    off = (jnp.arange(B, dtype=jnp.int32) * N)[:, None, None]
    fall = (F.astype(jnp.int32) + off).transpose(2, 0, 1).reshape(3 * NF)
    z = jnp.zeros((ZROWS,), jnp.float32)
    parts = _sc_call(vt, fall, z)
    summed = pl.pallas_call(
        _tc_add_body,
        out_shape=jax.ShapeDtypeStruct((3 * BN // 128, 128), jnp.float32),
    )(parts.reshape(NC, 3 * BN // 128, 128))
    return summed.reshape(3, BN).T
